# baseline (speedup 1.0000x reference)
import functools

import jax
import jax.numpy as jnp
import numpy as np
from jax import lax
from jax.experimental import pallas as pl
from jax.experimental.pallas import tpu as pltpu
from jax.experimental.pallas import tpu_sc as plsc

B, N, K = 4, 1024, 30
KP = 32
NE = B * N * KP
NAA = 20
NRBF = 16
BN_ROWS = 256
NB = 128
TE = NB * KP
LN_EPS = 1e-5

_BMAT = np.zeros((TE, NB), np.float32)
for _n in range(NB):
    _BMAT[_n * KP:(_n + 1) * KP, _n] = 1.0
_RMAT = np.zeros((NB, TE), np.float32)
for _n in range(NB):
    _RMAT[_n, _n * KP:_n * KP + K] = 1.0
_SINC = (6.28318298622592, -41.34143331806659, 81.59640430299906,
         -76.58159139136328, 41.20893327247536, -12.272487951343365)
_SIGMA = (22.0 - 2.0) / NRBF


def _bmat():
    return jnp.asarray(_BMAT)


def _rmat():
    return jnp.asarray(_RMAT)


def _ln(x):
    m = jnp.mean(x, axis=-1, keepdims=True)
    xc = x - m
    v = jnp.mean(xc * xc, axis=-1, keepdims=True)
    return xc / jnp.sqrt(v + LN_EPS)


def _mm(a, b):
    return jnp.dot(a, b, preferred_element_type=jnp.float32)


def _mmb(a, b):
    return jnp.dot(a.astype(jnp.bfloat16), b.astype(jnp.bfloat16),
                   preferred_element_type=jnp.float32)


def _sin2pi(t):
    s = t - jnp.floor(t + 0.5)
    s2 = s * s
    p = jnp.float32(_SINC[5])
    for c in (_SINC[4], _SINC[3], _SINC[2], _SINC[1], _SINC[0]):
        p = p * s2 + jnp.float32(c)
    return p * s


def _knn_body(cxr, cyr, czr, chr_, orr, cxa, cya, cza, cha, ora,
              idx_ref, dk_ref, same_ref, ar_ref):
    i = pl.program_id(0)
    b = i // (N // BN_ROWS)
    d2 = (cxr[...] - cxa[...][None, :]) ** 2
    d2 = d2 + (cyr[...] - cya[...][None, :]) ** 2
    d2 = d2 + (czr[...] - cza[...][None, :]) ** 2
    iota = lax.broadcasted_iota(jnp.int32, (BN_ROWS, N), 1)
    iota1 = lax.broadcasted_iota(jnp.int32, (1, N), 1)
    cmb_i = (ora[...] * 4.0 + cha[...]).astype(jnp.int32)[None, :]
    combo_row = iota1 * 4096 + cmb_i
    chr_i = chr_[...].astype(jnp.int32)
    orr_i = orr[...].astype(jnp.int32)
    big = jnp.int32(1 << 30)
    cols_i, cols_d, cols_s, cols_a = [], [], [], []
    cur = d2
    for _t in range(K):
        vmin = jnp.min(cur, axis=1, keepdims=True)
        ismin = cur == vmin
        sel = jnp.min(jnp.where(ismin, combo_row, big), axis=1, keepdims=True)
        cand = lax.shift_right_logical(sel, 12)
        cmb = sel & 4095
        oj = lax.shift_right_logical(cmb, 2)
        cj = cmb & 3
        cols_i.append(cand + b * N)
        cols_d.append(jnp.sqrt(vmin + 1e-8))
        cols_s.append((cj == chr_i).astype(jnp.float32))
        cols_a.append((oj < orr_i).astype(jnp.float32))
        cur = jnp.where(iota == cand, jnp.inf, cur)
    self_g = i * BN_ROWS + lax.broadcasted_iota(jnp.int32, (BN_ROWS, 1), 0)
    zcol = jnp.zeros((BN_ROWS, 1), jnp.float32)
    for _t in range(KP - K):
        cols_i.append(self_g)
        cols_d.append(zcol + 1e-4)
        cols_s.append(zcol + 1.0)
        cols_a.append(zcol)
    idx_ref[...] = jnp.concatenate(cols_i, axis=1)
    dk_ref[...] = jnp.concatenate(cols_d, axis=1)
    same_ref[...] = jnp.concatenate(cols_s, axis=1)
    ar_ref[...] = jnp.concatenate(cols_a, axis=1)


def _knn(C, chain_f, order_f):
    cx, cy, cz = C[:, :, 0].reshape(-1), C[:, :, 1].reshape(-1), C[:, :, 2].reshape(-1)
    grid = (B * N // BN_ROWS,)
    col = lambda: pl.BlockSpec((BN_ROWS, 1), lambda i: (i, 0))
    full = lambda: pl.BlockSpec((N,), lambda i: (i // (N // BN_ROWS),))
    out2 = lambda dt: jax.ShapeDtypeStruct((B * N, KP), dt)
    return pl.pallas_call(
        _knn_body,
        grid=grid,
        in_specs=[col(), col(), col(), col(), col(),
                  full(), full(), full(), full(), full()],
        out_specs=[pl.BlockSpec((BN_ROWS, KP), lambda i: (i, 0))] * 4,
        out_shape=[out2(jnp.int32), out2(jnp.float32), out2(jnp.float32),
                   out2(jnp.float32)],
    )(cx.reshape(-1, 1), cy.reshape(-1, 1), cz.reshape(-1, 1),
      chain_f.reshape(-1, 1), order_f.reshape(-1, 1),
      cx, cy, cz, chain_f.reshape(-1), order_f.reshape(-1))


def _feat_body(dk, same, lcol, emb, wl, centers, W_e, b_e, Rm, W1e, b1, W2, b2,
               e_ref, v_ref, s_ref):
    dkv = dk[...]
    c_row = centers[...][None, :]
    rbf = jnp.exp(-(((dkv - c_row) / _SIGMA) ** 2))
    wlf = _sin2pi(dkv / wl[...][None, :])
    We = W_e[...]
    e = _mmb(rbf, We[:NRBF]) + _mmb(wlf, We[NRBF:NRBF + 128])
    e = e + same[...] * We[NRBF + 128][None, :] + b_e[...][None, :]
    e_ref[...] = e
    pre = jax.nn.relu(_mmb(e, W1e[...]) + b1[...][None, :])
    t = _mmb(Rm[...], pre) * (1.0 / K)
    v_ref[...] = _ln(_mm(t, W2[...]) + b2[...][None, :])
    lv = lcol[...]
    lc = jnp.clip(lv, 0, NAA)
    oh = (lc == lax.broadcasted_iota(jnp.int32, (NB, NAA + 1), 1)).astype(jnp.float32)
    s_ref[...] = _mm(oh, emb[...]) * (lv >= 0).astype(jnp.float32)


def _featurize(dk_col, same_col, L, emb_S, wl, centers, W_e, b_e,
               W1e, b1, W2, b2):
    grid = (NE // TE,)
    col = pl.BlockSpec((TE, 1), lambda i: (i, 0))
    w128 = pl.BlockSpec((128, 128), lambda i: (0, 0))
    bspec = pl.BlockSpec((128,), lambda i: (0,))
    nspec = pl.BlockSpec((NB, 128), lambda i: (i, 0))
    return pl.pallas_call(
        _feat_body,
        grid=grid,
        in_specs=[col, col,
                  pl.BlockSpec((NB, 1), lambda i: (i, 0)),
                  pl.BlockSpec((NAA + 1, 128), lambda i: (0, 0)),
                  pl.BlockSpec((128,), lambda i: (0,)),
                  pl.BlockSpec((NRBF,), lambda i: (0,)),
                  pl.BlockSpec((NRBF + 129, 128), lambda i: (0, 0)),
                  bspec,
                  pl.BlockSpec((NB, TE), lambda i: (0, 0)),
                  w128, bspec, w128, bspec],
        out_specs=[pl.BlockSpec((TE, 128), lambda i: (i, 0)), nspec, nspec],
        out_shape=[jax.ShapeDtypeStruct((NE, 128), jnp.float32),
                   jax.ShapeDtypeStruct((B * N, 128), jnp.float32),
                   jax.ShapeDtypeStruct((B * N, 128), jnp.float32)],
    )(dk_col, same_col, L.reshape(-1, 1).astype(jnp.int32), emb_S,
      wl, centers, W_e, b_e, _rmat(), W1e, b1, W2, b2)


def _edge_node_body(e, gv, vb, Bm_r, Rm_r, eW1, eb1, eW2, eb2, W1, b1, W2, b2,
                    e_out, v_out):
    Bm = Bm_r[...]
    Rm = Rm_r[...]
    ev, gvv, vbv = e[...], gv[...], vb[...]
    eW1v = eW1[...]
    a_e = _mmb(Bm, _mm(vbv, eW1v[0:128]))
    mid = jax.nn.relu(a_e + _mmb(gvv, eW1v[128:256]) + _mmb(ev, eW1v[256:384])
                      + eb1[...][None, :])
    me = _mmb(mid, eW2[...]) + eb2[...][None, :]
    e1 = _ln(ev + me)
    e_out[...] = e1
    W1v = W1[...]
    a_n = _mmb(Bm, _mm(vbv, W1v[0:128]))
    pre = jax.nn.relu(a_n + _mmb(gvv, W1v[128:256]) + _mmb(e1, W1v[256:384])
                      + b1[...][None, :])
    t = _mm(_mmb(Rm, pre) * (1.0 / K), W2[...]) + b2[...][None, :]
    v_out[...] = _ln(vbv + t)


def _edge_then_node(E, GV, V, eW1, eb1, eW2, eb2, W1, b1, W2, b2):
    grid = (B * N // NB,)
    espec = pl.BlockSpec((TE, 128), lambda i: (i, 0))
    vspec = pl.BlockSpec((NB, 128), lambda i: (i, 0))
    w384 = pl.BlockSpec((384, 128), lambda i: (0, 0))
    w128 = pl.BlockSpec((128, 128), lambda i: (0, 0))
    bspec = pl.BlockSpec((128,), lambda i: (0,))
    return pl.pallas_call(
        _edge_node_body,
        grid=grid,
        in_specs=[espec, espec, vspec,
                  pl.BlockSpec((TE, NB), lambda i: (0, 0)),
                  pl.BlockSpec((NB, TE), lambda i: (0, 0)),
                  w384, bspec, w128, bspec,
                  w384, bspec, w128, bspec],
        out_specs=[espec, vspec],
        out_shape=[jax.ShapeDtypeStruct((NE, 128), jnp.float32),
                   jax.ShapeDtypeStruct((B * N, 128), jnp.float32)],
    )(E, GV, V, _bmat(), _rmat(), eW1, eb1, eW2, eb2, W1, b1, W2, b2)


def _dec_body(e, gv, gs, arc, vb, Bm_r, Rm_r, eW1, eb1, eW2, eb2,
              W1v3, W1e3, W1j3, W1s3, b1_3, W2_3, b2_3, W_out, b_out, o_ref):
    Bm = Bm_r[...]
    Rm = Rm_r[...]
    ev, gvv, vbv = e[...], gv[...], vb[...]
    eW1v = eW1[...]
    a_e = _mmb(Bm, _mm(vbv, eW1v[0:128]))
    mid = jax.nn.relu(a_e + _mmb(gvv, eW1v[128:256]) + _mmb(ev, eW1v[256:384])
                      + eb1[...][None, :])
    e3 = _ln(ev + _mmb(mid, eW2[...]) + eb2[...][None, :])
    sj = gs[...] * arc[...]
    vn = vbv
    for l in range(3):
        a_n = _mmb(Bm, _mm(vn, W1v3[l]))
        pre = jax.nn.relu(a_n + _mmb(e3, W1e3[l]) + _mmb(gvv, W1j3[l])
                          + _mmb(sj, W1s3[l]) + b1_3[l][None, :])
        t = _mm(_mmb(Rm, pre) * (1.0 / K), W2_3[l]) + b2_3[l][None, :]
        vn = _ln(vn + t)
    o_ref[...] = _mm(vn, W_out[...]) + b_out[...][None, :]


def _decoder(E, GV, GS, ar_col, V, eW1, eb1, eW2, eb2,
             W1v3, W1e3, W1j3, W1s3, b1_3, W2_3, b2_3, W_out, b_out):
    grid = (B * N // NB,)
    espec = pl.BlockSpec((TE, 128), lambda i: (i, 0))
    w3 = pl.BlockSpec((3, 128, 128), lambda i: (0, 0, 0))
    b3 = pl.BlockSpec((3, 128), lambda i: (0, 0))
    return pl.pallas_call(
        _dec_body,
        grid=grid,
        in_specs=[espec, espec, espec,
                  pl.BlockSpec((TE, 1), lambda i: (i, 0)),
                  pl.BlockSpec((NB, 128), lambda i: (i, 0)),
                  pl.BlockSpec((TE, NB), lambda i: (0, 0)),
                  pl.BlockSpec((NB, TE), lambda i: (0, 0)),
                  pl.BlockSpec((384, 128), lambda i: (0, 0)),
                  pl.BlockSpec((128,), lambda i: (0,)),
                  pl.BlockSpec((128, 128), lambda i: (0, 0)),
                  pl.BlockSpec((128,), lambda i: (0,)),
                  w3, w3, w3, w3, b3, w3, b3,
                  pl.BlockSpec((128, NAA), lambda i: (0, 0)),
                  pl.BlockSpec((NAA,), lambda i: (0,))],
        out_specs=pl.BlockSpec((NB, NAA), lambda i: (i, 0)),
        out_shape=jax.ShapeDtypeStruct((B * N, NAA), jnp.float32),
    )(E, GV, GS, ar_col, V, _bmat(), _rmat(), eW1, eb1, eW2, eb2,
      W1v3, W1e3, W1j3, W1s3, b1_3, W2_3, b2_3, W_out, b_out)


_SC_NC, _SC_NS = 2, 16
_SC_NW = _SC_NC * _SC_NS
_IDX_ROWS = NE // 128
_ROWS_PER_W = _IDX_ROWS // _SC_NW
_SC_NBUF = 4


def _sc_gather(idx2d, table):
    mesh = plsc.VectorSubcoreMesh(core_axis_name="c", subcore_axis_name="s")
    scratch = []
    for _j in range(_SC_NBUF):
        scratch.append(pltpu.VMEM((128,), jnp.int32))
        scratch.append(pltpu.VMEM((128, 128), jnp.float32))
        scratch.append(pltpu.SemaphoreType.DMA)

    @functools.partial(
        pl.kernel, mesh=mesh,
        out_type=jax.ShapeDtypeStruct((NE, 128), jnp.float32),
        scratch_types=scratch,
    )
    def k(idx_hbm, table_hbm, out_hbm, *bufs):
        idxv = bufs[0::3]
        rows = bufs[1::3]
        sems = bufs[2::3]
        wid = lax.axis_index("s") * _SC_NC + lax.axis_index("c")
        row0 = wid * _ROWS_PER_W

        def body(step, carry):
            base = row0 + step * _SC_NBUF
            for j in range(_SC_NBUF):
                pltpu.sync_copy(idx_hbm.at[base + j], idxv[j])
            cps = [pltpu.async_copy(table_hbm.at[idxv[j]], rows[j], sems[j])
                   for j in range(_SC_NBUF)]
            for j in range(_SC_NBUF):
                cps[j].wait()
                r = base + j
                pltpu.sync_copy(
                    rows[j], out_hbm.at[pl.ds(pl.multiple_of(r * 128, 128), 128)])
            return carry

        lax.fori_loop(0, _ROWS_PER_W // _SC_NBUF, body, 0)

    return k(idx2d, table)


def _sc_gather2(idx2d, table_a, table_b):
    mesh = plsc.VectorSubcoreMesh(core_axis_name="c", subcore_axis_name="s")
    scratch = []
    for _j in range(_SC_NBUF):
        scratch.append(pltpu.VMEM((128,), jnp.int32))
        scratch.append(pltpu.VMEM((128, 128), jnp.float32))
        scratch.append(pltpu.SemaphoreType.DMA)

    @functools.partial(
        pl.kernel, mesh=mesh,
        out_type=[jax.ShapeDtypeStruct((NE, 128), jnp.float32),
                  jax.ShapeDtypeStruct((NE, 128), jnp.float32)],
        scratch_types=scratch,
    )
    def k(idx_hbm, ta_hbm, tb_hbm, oa_hbm, ob_hbm, *bufs):
        idxv = bufs[0::3]
        rows = bufs[1::3]
        sems = bufs[2::3]
        wid = lax.axis_index("s") * _SC_NC + lax.axis_index("c")
        row0 = wid * _ROWS_PER_W
        npair = _SC_NBUF // 2

        def body(step, carry):
            base = row0 + step * npair
            for j in range(npair):
                pltpu.sync_copy(idx_hbm.at[base + j], idxv[j])
            cps = []
            for j in range(npair):
                cps.append(pltpu.async_copy(ta_hbm.at[idxv[j]],
                                            rows[2 * j], sems[2 * j]))
                cps.append(pltpu.async_copy(tb_hbm.at[idxv[j]],
                                            rows[2 * j + 1], sems[2 * j + 1]))
            for j in range(npair):
                r = base + j
                off = pl.ds(pl.multiple_of(r * 128, 128), 128)
                cps[2 * j].wait()
                pltpu.sync_copy(rows[2 * j], oa_hbm.at[off])
                cps[2 * j + 1].wait()
                pltpu.sync_copy(rows[2 * j + 1], ob_hbm.at[off])
            return carry

        lax.fori_loop(0, _ROWS_PER_W // npair, body, 0)

    return k(idx2d, table_a, table_b)


def kernel(C, L, chain_idxs, decoding_order, wl, emb_S, W_e, b_e,
           enc_W1, enc_b1, enc_W2, enc_b2, enc_eW1, enc_eb1, enc_eW2, enc_eb2,
           dec_W1, dec_b1, dec_W2, dec_b2, W_out, b_out):
    chain_f = chain_idxs.astype(jnp.float32)
    order_f = decoding_order.astype(jnp.float32)

    gidx2, dk2, same2, ar2 = _knn(C, chain_f, order_f)
    idx2d = gidx2.reshape(_IDX_ROWS, 128)
    dk_col = dk2.reshape(NE, 1)
    same_col = same2.reshape(NE, 1)
    ar_col = ar2.reshape(NE, 1)

    centers = jnp.linspace(2.0, 22.0, NRBF)
    E0, V1, S = _featurize(dk_col, same_col, L, emb_S, wl, centers, W_e, b_e,
                           enc_W1[0, 256:384], enc_b1[0], enc_W2[0], enc_b2[0])

    GV1 = _sc_gather(idx2d, V1)
    E1, V2 = _edge_then_node(E0, GV1, V1,
                             enc_eW1[0], enc_eb1[0], enc_eW2[0], enc_eb2[0],
                             enc_W1[1], enc_b1[1], enc_W2[1], enc_b2[1])
    GV2 = _sc_gather(idx2d, V2)
    E2, V3 = _edge_then_node(E1, GV2, V2,
                             enc_eW1[1], enc_eb1[1], enc_eW2[1], enc_eb2[1],
                             enc_W1[2], enc_b1[2], enc_W2[2], enc_b2[2])
    GV3, GS = _sc_gather2(idx2d, V3, S)

    out = _decoder(E2, GV3, GS, ar_col, V3,
                   enc_eW1[2], enc_eb1[2], enc_eW2[2], enc_eb2[2],
                   dec_W1[:, 0:128, :], dec_W1[:, 128:256, :],
                   dec_W1[:, 256:384, :], dec_W1[:, 384:512, :],
                   dec_b1, dec_W2, dec_b2, W_out, b_out)
    return out.reshape(B, N, NAA)


# speedup vs baseline: 1.0308x; 1.0308x over previous
import functools

import jax
import jax.numpy as jnp
import numpy as np
from jax import lax
from jax.experimental import pallas as pl
from jax.experimental.pallas import tpu as pltpu
from jax.experimental.pallas import tpu_sc as plsc

B, N, K = 4, 1024, 30
KP = 32
NE = B * N * KP
NAA = 20
NRBF = 16
BN_ROWS = 256
NB = 128
TE = NB * KP
LN_EPS = 1e-5

_BMAT = np.zeros((TE, NB), np.float32)
for _n in range(NB):
    _BMAT[_n * KP:(_n + 1) * KP, _n] = 1.0
_RMAT = np.zeros((NB, TE), np.float32)
for _n in range(NB):
    _RMAT[_n, _n * KP:_n * KP + K] = 1.0
_SINC = (6.28318298622592, -41.34143331806659, 81.59640430299906,
         -76.58159139136328, 41.20893327247536, -12.272487951343365)
_SIGMA = (22.0 - 2.0) / NRBF


def _bmat():
    return jnp.asarray(_BMAT)


def _rmat():
    return jnp.asarray(_RMAT)


def _ln(x):
    m = jnp.mean(x, axis=-1, keepdims=True)
    xc = x - m
    v = jnp.mean(xc * xc, axis=-1, keepdims=True)
    return xc / jnp.sqrt(v + LN_EPS)


def _mm(a, b):
    return jnp.dot(a, b, preferred_element_type=jnp.float32)


def _mmb(a, b):
    return jnp.dot(a.astype(jnp.bfloat16), b.astype(jnp.bfloat16),
                   preferred_element_type=jnp.float32)


def _sin2pi(t):
    s = t - jnp.floor(t + 0.5)
    s2 = s * s
    p = jnp.float32(_SINC[5])
    for c in (_SINC[4], _SINC[3], _SINC[2], _SINC[1], _SINC[0]):
        p = p * s2 + jnp.float32(c)
    return p * s


def _pack_bf16(v):
    u = lax.bitcast_convert_type(v.astype(jnp.bfloat16), jnp.uint16)
    lo = u[:, :64].astype(jnp.uint32)
    hi = u[:, 64:].astype(jnp.uint32)
    return lax.bitcast_convert_type(lo | (hi << 16), jnp.int32)


def _unpack_bf16(p):
    u = lax.bitcast_convert_type(p, jnp.uint32)
    lo = (u & jnp.uint32(0xFFFF)).astype(jnp.uint16)
    hi = lax.shift_right_logical(u, jnp.uint32(16)).astype(jnp.uint16)
    return jnp.concatenate(
        [lax.bitcast_convert_type(lo, jnp.bfloat16),
         lax.bitcast_convert_type(hi, jnp.bfloat16)], axis=1)


def _knn_body(cxr, cyr, czr, chr_, orr, cxa, cya, cza, cha, ora,
              idx_ref, dk_ref, same_ref, ar_ref):
    i = pl.program_id(0)
    b = i // (N // BN_ROWS)
    d2 = (cxr[...] - cxa[...][None, :]) ** 2
    d2 = d2 + (cyr[...] - cya[...][None, :]) ** 2
    d2 = d2 + (czr[...] - cza[...][None, :]) ** 2
    iota = lax.broadcasted_iota(jnp.int32, (BN_ROWS, N), 1)
    iota1 = lax.broadcasted_iota(jnp.int32, (1, N), 1)
    cmb_i = (ora[...] * 4.0 + cha[...]).astype(jnp.int32)[None, :]
    combo_row = iota1 * 4096 + cmb_i
    chr_i = chr_[...].astype(jnp.int32)
    orr_i = orr[...].astype(jnp.int32)
    big = jnp.int32(1 << 30)
    cols_i, cols_d, cols_s, cols_a = [], [], [], []
    cur = d2
    for _t in range(K):
        vmin = jnp.min(cur, axis=1, keepdims=True)
        ismin = cur == vmin
        sel = jnp.min(jnp.where(ismin, combo_row, big), axis=1, keepdims=True)
        cand = lax.shift_right_logical(sel, 12)
        cmb = sel & 4095
        oj = lax.shift_right_logical(cmb, 2)
        cj = cmb & 3
        cols_i.append(cand + b * N)
        cols_d.append(jnp.sqrt(vmin + 1e-8))
        cols_s.append((cj == chr_i).astype(jnp.float32))
        cols_a.append((oj < orr_i).astype(jnp.float32))
        cur = jnp.where(iota == cand, jnp.inf, cur)
    self_g = i * BN_ROWS + lax.broadcasted_iota(jnp.int32, (BN_ROWS, 1), 0)
    zcol = jnp.zeros((BN_ROWS, 1), jnp.float32)
    for _t in range(KP - K):
        cols_i.append(self_g)
        cols_d.append(zcol + 1e-4)
        cols_s.append(zcol + 1.0)
        cols_a.append(zcol)
    idx_ref[...] = jnp.concatenate(cols_i, axis=1)
    dk_ref[...] = jnp.concatenate(cols_d, axis=1)
    same_ref[...] = jnp.concatenate(cols_s, axis=1)
    ar_ref[...] = jnp.concatenate(cols_a, axis=1)


def _knn(C, chain_f, order_f):
    cx, cy, cz = C[:, :, 0].reshape(-1), C[:, :, 1].reshape(-1), C[:, :, 2].reshape(-1)
    grid = (B * N // BN_ROWS,)
    col = lambda: pl.BlockSpec((BN_ROWS, 1), lambda i: (i, 0))
    full = lambda: pl.BlockSpec((N,), lambda i: (i // (N // BN_ROWS),))
    out2 = lambda dt: jax.ShapeDtypeStruct((B * N, KP), dt)
    return pl.pallas_call(
        _knn_body,
        grid=grid,
        in_specs=[col(), col(), col(), col(), col(),
                  full(), full(), full(), full(), full()],
        out_specs=[pl.BlockSpec((BN_ROWS, KP), lambda i: (i, 0))] * 4,
        out_shape=[out2(jnp.int32), out2(jnp.float32), out2(jnp.float32),
                   out2(jnp.float32)],
    )(cx.reshape(-1, 1), cy.reshape(-1, 1), cz.reshape(-1, 1),
      chain_f.reshape(-1, 1), order_f.reshape(-1, 1),
      cx, cy, cz, chain_f.reshape(-1), order_f.reshape(-1))


def _feat_body(dk, same, lcol, emb, wl, centers, W_e, b_e, Rm, W1e, b1, W2, b2,
               e_ref, v_ref, vp_ref, sp_ref):
    dkv = dk[...]
    c_row = centers[...][None, :]
    rbf = jnp.exp(-(((dkv - c_row) / _SIGMA) ** 2))
    wlf = _sin2pi(dkv / wl[...][None, :])
    We = W_e[...]
    e = _mmb(rbf, We[:NRBF]) + _mmb(wlf, We[NRBF:NRBF + 128])
    e = e + same[...] * We[NRBF + 128][None, :] + b_e[...][None, :]
    e_ref[...] = e
    pre = jax.nn.relu(_mmb(e, W1e[...]) + b1[...][None, :])
    t = _mmb(Rm[...], pre) * (1.0 / K)
    v1 = _ln(_mm(t, W2[...]) + b2[...][None, :])
    v_ref[...] = v1
    vp_ref[...] = _pack_bf16(v1)
    lv = lcol[...]
    lc = jnp.clip(lv, 0, NAA)
    oh = (lc == lax.broadcasted_iota(jnp.int32, (NB, NAA + 1), 1)).astype(jnp.float32)
    s = _mm(oh, emb[...]) * (lv >= 0).astype(jnp.float32)
    sp_ref[...] = _pack_bf16(s)


def _featurize(dk_col, same_col, L, emb_S, wl, centers, W_e, b_e,
               W1e, b1, W2, b2):
    grid = (NE // TE,)
    col = pl.BlockSpec((TE, 1), lambda i: (i, 0))
    w128 = pl.BlockSpec((128, 128), lambda i: (0, 0))
    bspec = pl.BlockSpec((128,), lambda i: (0,))
    nspec = pl.BlockSpec((NB, 128), lambda i: (i, 0))
    return pl.pallas_call(
        _feat_body,
        grid=grid,
        in_specs=[col, col,
                  pl.BlockSpec((NB, 1), lambda i: (i, 0)),
                  pl.BlockSpec((NAA + 1, 128), lambda i: (0, 0)),
                  pl.BlockSpec((128,), lambda i: (0,)),
                  pl.BlockSpec((NRBF,), lambda i: (0,)),
                  pl.BlockSpec((NRBF + 129, 128), lambda i: (0, 0)),
                  bspec,
                  pl.BlockSpec((NB, TE), lambda i: (0, 0)),
                  w128, bspec, w128, bspec],
        out_specs=[pl.BlockSpec((TE, 128), lambda i: (i, 0)), nspec,
                   pl.BlockSpec((NB, 64), lambda i: (i, 0)),
                   pl.BlockSpec((NB, 64), lambda i: (i, 0))],
        out_shape=[jax.ShapeDtypeStruct((NE, 128), jnp.float32),
                   jax.ShapeDtypeStruct((B * N, 128), jnp.float32),
                   jax.ShapeDtypeStruct((B * N, 64), jnp.int32),
                   jax.ShapeDtypeStruct((B * N, 64), jnp.int32)],
    )(dk_col, same_col, L.reshape(-1, 1).astype(jnp.int32), emb_S,
      wl, centers, W_e, b_e, _rmat(), W1e, b1, W2, b2)


def _edge_node_body(e, gv, vb, Bm_r, Rm_r, eW1, eb1, eW2, eb2, W1, b1, W2, b2,
                    e_out, v_out, vp_out):
    Bm = Bm_r[...]
    Rm = Rm_r[...]
    ev, gvv, vbv = e[...], gv[...], vb[...]
    eW1v = eW1[...]
    a_e = _mmb(Bm, _mm(vbv, eW1v[0:128]))
    mid = jax.nn.relu(a_e + _mmb(gvv, eW1v[128:256]) + _mmb(ev, eW1v[256:384])
                      + eb1[...][None, :])
    me = _mmb(mid, eW2[...]) + eb2[...][None, :]
    e1 = _ln(ev + me)
    e_out[...] = e1
    W1v = W1[...]
    a_n = _mmb(Bm, _mm(vbv, W1v[0:128]))
    pre = jax.nn.relu(a_n + _mmb(gvv, W1v[128:256]) + _mmb(e1, W1v[256:384])
                      + b1[...][None, :])
    t = _mm(_mmb(Rm, pre) * (1.0 / K), W2[...]) + b2[...][None, :]
    v2 = _ln(vbv + t)
    v_out[...] = v2
    vp_out[...] = _pack_bf16(v2)


def _edge_then_node(E, GV, V, eW1, eb1, eW2, eb2, W1, b1, W2, b2):
    grid = (B * N // NB,)
    espec = pl.BlockSpec((TE, 128), lambda i: (i, 0))
    vspec = pl.BlockSpec((NB, 128), lambda i: (i, 0))
    w384 = pl.BlockSpec((384, 128), lambda i: (0, 0))
    w128 = pl.BlockSpec((128, 128), lambda i: (0, 0))
    bspec = pl.BlockSpec((128,), lambda i: (0,))
    return pl.pallas_call(
        _edge_node_body,
        grid=grid,
        in_specs=[espec, espec, vspec,
                  pl.BlockSpec((TE, NB), lambda i: (0, 0)),
                  pl.BlockSpec((NB, TE), lambda i: (0, 0)),
                  w384, bspec, w128, bspec,
                  w384, bspec, w128, bspec],
        out_specs=[espec, vspec, pl.BlockSpec((NB, 64), lambda i: (i, 0))],
        out_shape=[jax.ShapeDtypeStruct((NE, 128), jnp.float32),
                   jax.ShapeDtypeStruct((B * N, 128), jnp.float32),
                   jax.ShapeDtypeStruct((B * N, 64), jnp.int32)],
    )(E, GV, V, _bmat(), _rmat(), eW1, eb1, eW2, eb2, W1, b1, W2, b2)


def _dec_body(e, g, arc, vb, Bm_r, Rm_r, eW1, eb1, eW2, eb2,
              W1v3, W1e3, W1j3, W1s3, b1_3, W2_3, b2_3, W_out, b_out, o_ref):
    Bm = Bm_r[...]
    Rm = Rm_r[...]
    ev, vbv = e[...], vb[...]
    gval = g[...]
    gvv = _unpack_bf16(gval[:, :64])
    eW1v = eW1[...]
    a_e = _mmb(Bm, _mm(vbv, eW1v[0:128]))
    mid = jax.nn.relu(a_e + _mmb(gvv, eW1v[128:256]) + _mmb(ev, eW1v[256:384])
                      + eb1[...][None, :])
    e3 = _ln(ev + _mmb(mid, eW2[...]) + eb2[...][None, :])
    sj = _unpack_bf16(gval[:, 64:]).astype(jnp.float32) * arc[...]
    vn = vbv
    for l in range(3):
        a_n = _mmb(Bm, _mm(vn, W1v3[l]))
        pre = jax.nn.relu(a_n + _mmb(e3, W1e3[l]) + _mmb(gvv, W1j3[l])
                          + _mmb(sj, W1s3[l]) + b1_3[l][None, :])
        t = _mm(_mmb(Rm, pre) * (1.0 / K), W2_3[l]) + b2_3[l][None, :]
        vn = _ln(vn + t)
    o_ref[...] = _mm(vn, W_out[...]) + b_out[...][None, :]


def _decoder(E, G, ar_col, V, eW1, eb1, eW2, eb2,
             W1v3, W1e3, W1j3, W1s3, b1_3, W2_3, b2_3, W_out, b_out):
    grid = (B * N // NB,)
    espec = pl.BlockSpec((TE, 128), lambda i: (i, 0))
    w3 = pl.BlockSpec((3, 128, 128), lambda i: (0, 0, 0))
    b3 = pl.BlockSpec((3, 128), lambda i: (0, 0))
    return pl.pallas_call(
        _dec_body,
        grid=grid,
        in_specs=[espec, espec,
                  pl.BlockSpec((TE, 1), lambda i: (i, 0)),
                  pl.BlockSpec((NB, 128), lambda i: (i, 0)),
                  pl.BlockSpec((TE, NB), lambda i: (0, 0)),
                  pl.BlockSpec((NB, TE), lambda i: (0, 0)),
                  pl.BlockSpec((384, 128), lambda i: (0, 0)),
                  pl.BlockSpec((128,), lambda i: (0,)),
                  pl.BlockSpec((128, 128), lambda i: (0, 0)),
                  pl.BlockSpec((128,), lambda i: (0,)),
                  w3, w3, w3, w3, b3, w3, b3,
                  pl.BlockSpec((128, NAA), lambda i: (0, 0)),
                  pl.BlockSpec((NAA,), lambda i: (0,))],
        out_specs=pl.BlockSpec((NB, NAA), lambda i: (i, 0)),
        out_shape=jax.ShapeDtypeStruct((B * N, NAA), jnp.float32),
    )(E, G, ar_col, V, _bmat(), _rmat(), eW1, eb1, eW2, eb2,
      W1v3, W1e3, W1j3, W1s3, b1_3, W2_3, b2_3, W_out, b_out)


_SC_NC, _SC_NS = 2, 16
_SC_NW = _SC_NC * _SC_NS
_IDX_ROWS = NE // 128
_ROWS_PER_W = _IDX_ROWS // _SC_NW
_SC_NBUF = 4


def _sc_gather(idx2d, table):
    mesh = plsc.VectorSubcoreMesh(core_axis_name="c", subcore_axis_name="s")
    scratch = []
    for _j in range(_SC_NBUF):
        scratch.append(pltpu.VMEM((128,), jnp.int32))
        scratch.append(pltpu.VMEM((128, 128), jnp.float32))
        scratch.append(pltpu.SemaphoreType.DMA)

    @functools.partial(
        pl.kernel, mesh=mesh,
        out_type=jax.ShapeDtypeStruct((NE, 128), jnp.float32),
        scratch_types=scratch,
    )
    def k(idx_hbm, table_hbm, out_hbm, *bufs):
        idxv = bufs[0::3]
        rows = bufs[1::3]
        sems = bufs[2::3]
        wid = lax.axis_index("s") * _SC_NC + lax.axis_index("c")
        row0 = wid * _ROWS_PER_W

        def body(step, carry):
            base = row0 + step * _SC_NBUF
            for j in range(_SC_NBUF):
                pltpu.sync_copy(idx_hbm.at[base + j], idxv[j])
            cps = [pltpu.async_copy(table_hbm.at[idxv[j]], rows[j], sems[j])
                   for j in range(_SC_NBUF)]
            for j in range(_SC_NBUF):
                cps[j].wait()
                r = base + j
                pltpu.sync_copy(
                    rows[j], out_hbm.at[pl.ds(pl.multiple_of(r * 128, 128), 128)])
            return carry

        lax.fori_loop(0, _ROWS_PER_W // _SC_NBUF, body, 0)

    return k(idx2d, table)


def _sc_gather_packed(idx2d, table):
    mesh = plsc.VectorSubcoreMesh(core_axis_name="c", subcore_axis_name="s")
    scratch = []
    for _j in range(_SC_NBUF):
        scratch.append(pltpu.VMEM((128,), jnp.int32))
        scratch.append(pltpu.VMEM((128, 128), jnp.int32))
        scratch.append(pltpu.SemaphoreType.DMA)

    @functools.partial(
        pl.kernel, mesh=mesh,
        out_type=jax.ShapeDtypeStruct((NE, 128), jnp.int32),
        scratch_types=scratch,
    )
    def k(idx_hbm, table_hbm, out_hbm, *bufs):
        idxv = bufs[0::3]
        rows = bufs[1::3]
        sems = bufs[2::3]
        wid = lax.axis_index("s") * _SC_NC + lax.axis_index("c")
        row0 = wid * _ROWS_PER_W

        def body(step, carry):
            base = row0 + step * _SC_NBUF
            for j in range(_SC_NBUF):
                pltpu.sync_copy(idx_hbm.at[base + j], idxv[j])
            cps = [pltpu.async_copy(table_hbm.at[idxv[j]], rows[j], sems[j])
                   for j in range(_SC_NBUF)]
            for j in range(_SC_NBUF):
                cps[j].wait()
                r = base + j
                pltpu.sync_copy(
                    rows[j], out_hbm.at[pl.ds(pl.multiple_of(r * 128, 128), 128)])
            return carry

        lax.fori_loop(0, _ROWS_PER_W // _SC_NBUF, body, 0)

    return k(idx2d, table)


def kernel(C, L, chain_idxs, decoding_order, wl, emb_S, W_e, b_e,
           enc_W1, enc_b1, enc_W2, enc_b2, enc_eW1, enc_eb1, enc_eW2, enc_eb2,
           dec_W1, dec_b1, dec_W2, dec_b2, W_out, b_out):
    chain_f = chain_idxs.astype(jnp.float32)
    order_f = decoding_order.astype(jnp.float32)

    gidx2, dk2, same2, ar2 = _knn(C, chain_f, order_f)
    idx2d = gidx2.reshape(_IDX_ROWS, 128)
    dk_col = dk2.reshape(NE, 1)
    same_col = same2.reshape(NE, 1)
    ar_col = ar2.reshape(NE, 1)

    centers = jnp.linspace(2.0, 22.0, NRBF)
    E0, V1, V1p, Sp = _featurize(dk_col, same_col, L, emb_S, wl, centers,
                                 W_e, b_e, enc_W1[0, 256:384], enc_b1[0],
                                 enc_W2[0], enc_b2[0])

    GV1 = _sc_gather(idx2d, V1)
    E1, V2, V2p = _edge_then_node(E0, GV1, V1,
                                  enc_eW1[0], enc_eb1[0], enc_eW2[0], enc_eb2[0],
                                  enc_W1[1], enc_b1[1], enc_W2[1], enc_b2[1])
    GV2 = _sc_gather(idx2d, V2)
    E2, V3, V3p = _edge_then_node(E1, GV2, V2,
                                  enc_eW1[1], enc_eb1[1], enc_eW2[1], enc_eb2[1],
                                  enc_W1[2], enc_b1[2], enc_W2[2], enc_b2[2])
    G = _sc_gather_packed(idx2d, jnp.concatenate([V3p, Sp], axis=1))

    out = _decoder(E2, G, ar_col, V3,
                   enc_eW1[2], enc_eb1[2], enc_eW2[2], enc_eb2[2],
                   dec_W1[:, 0:128, :], dec_W1[:, 128:256, :],
                   dec_W1[:, 256:384, :], dec_W1[:, 384:512, :],
                   dec_b1, dec_W2, dec_b2, W_out, b_out)
    return out.reshape(B, N, NAA)
